# baseline (speedup 1.0000x reference)
import functools

import jax
import jax.numpy as jnp
from jax import lax
from jax.experimental import pallas as pl
from jax.experimental.pallas import tpu as pltpu
from jax.experimental.pallas import tpu_sc as plsc


def _gather_call(table2, idx):
    V2, D2 = table2.shape
    B = idx.shape[0]
    NC, NS, L = 2, 16, 16
    NW = NC * NS
    b_per_w = B // NW

    mesh = plsc.VectorSubcoreMesh(core_axis_name="c", subcore_axis_name="s")

    @functools.partial(
        pl.kernel,
        mesh=mesh,
        out_type=jax.ShapeDtypeStruct((B, D2), jnp.float32),
        scratch_types=[
            pltpu.VMEM((b_per_w,), jnp.int32),
            pltpu.VMEM((b_per_w,), jnp.int32),
            pltpu.VMEM((b_per_w, D2), jnp.float32),
            pltpu.SemaphoreType.DMA,
        ],
    )
    def k(tbl_hbm, idx_hbm, out_hbm, idx_v, row_v, rows_v, sem):
        wid = lax.axis_index("s") * NC + lax.axis_index("c")
        base = wid * b_per_w

        pltpu.sync_copy(idx_hbm.at[pl.ds(base, b_per_w)], idx_v)

        def halve(c, _):
            row_v[pl.ds(c * L, L)] = idx_v[pl.ds(c * L, L)] >> 1
            return ()

        lax.fori_loop(0, b_per_w // L, halve, (), unroll=4)

        pltpu.async_copy(tbl_hbm.at[row_v], rows_v, sem).wait()
        pltpu.sync_copy(rows_v, out_hbm.at[pl.ds(base, b_per_w)])

    return k(table2, idx)


@jax.jit
def kernel(x, policy):
    idx = x.astype(jnp.int32)
    n_states, n_actions = policy.shape
    table2 = policy.reshape(n_states // 2, 2 * n_actions)
    pairs = _gather_call(table2, idx)
    return jnp.where(
        (idx[:, None] & 1) == 1, pairs[:, n_actions:], pairs[:, :n_actions]
    )


# speedup vs baseline: 1.0003x; 1.0003x over previous
import functools

import jax
import jax.numpy as jnp
from jax import lax
from jax.experimental import pallas as pl
from jax.experimental.pallas import tpu as pltpu
from jax.experimental.pallas import tpu_sc as plsc
from jax.experimental.layout import Format, Layout, with_layout_constraint


def _gather_call(table2, idx):
    V2, D2 = table2.shape
    B = idx.shape[0]
    NC, NS, L = 2, 16, 16
    NW = NC * NS
    b_per_w = B // NW

    mesh = plsc.VectorSubcoreMesh(core_axis_name="c", subcore_axis_name="s")

    @functools.partial(
        pl.kernel,
        mesh=mesh,
        out_type=jax.ShapeDtypeStruct((B, D2), jnp.float32),
        scratch_types=[
            pltpu.VMEM((b_per_w,), jnp.int32),
            pltpu.VMEM((b_per_w,), jnp.int32),
            pltpu.VMEM((b_per_w, D2), jnp.float32),
            pltpu.SemaphoreType.DMA,
        ],
    )
    def k(tbl_hbm, idx_hbm, out_hbm, idx_v, row_v, rows_v, sem):
        wid = lax.axis_index("s") * NC + lax.axis_index("c")
        base = wid * b_per_w

        pltpu.sync_copy(idx_hbm.at[pl.ds(base, b_per_w)], idx_v)

        def halve(c, _):
            row_v[pl.ds(c * L, L)] = idx_v[pl.ds(c * L, L)] >> 1
            return ()

        lax.fori_loop(0, b_per_w // L, halve, (), unroll=4)

        pltpu.async_copy(tbl_hbm.at[row_v], rows_v, sem).wait()
        pltpu.sync_copy(rows_v, out_hbm.at[pl.ds(base, b_per_w)])

    return k(table2, idx)


@jax.jit
def kernel(x, policy):
    idx = x.astype(jnp.int32)
    n_states, n_actions = policy.shape
    plin = with_layout_constraint(
        policy, Layout(major_to_minor=(0, 1), tiling=((16,),))
    )
    table2 = plin.reshape(n_states // 2, 2 * n_actions)
    pairs = _gather_call(table2, idx)
    return jnp.where(
        (idx[:, None] & 1) == 1, pairs[:, n_actions:], pairs[:, :n_actions]
    )


# speedup vs baseline: 1.1443x; 1.1440x over previous
import functools

import jax
import jax.numpy as jnp
from jax import lax
from jax.experimental import pallas as pl
from jax.experimental.pallas import tpu as pltpu
from jax.experimental.pallas import tpu_sc as plsc


def _tc_relayout(table_t):
    D, V = table_t.shape
    W = 1920
    grid = -(-V // W)

    def body(in_ref, out_ref):
        a = in_ref[...].T
        out_ref[...] = jnp.concatenate([a, a], axis=1)

    return pl.pallas_call(
        body,
        grid=(grid,),
        in_specs=[pl.BlockSpec((D, W), lambda i: (0, i))],
        out_specs=pl.BlockSpec((W, 2 * D), lambda i: (i, 0)),
        out_shape=jax.ShapeDtypeStruct((V, 2 * D), jnp.float32),
    )(table_t)


def _gather_call(table2, idx, D):
    V2, D2 = table2.shape
    B = idx.shape[0]
    NC, NS = 2, 16
    NW = NC * NS
    b_per_w = B // NW

    mesh = plsc.VectorSubcoreMesh(core_axis_name="c", subcore_axis_name="s")

    @functools.partial(
        pl.kernel,
        mesh=mesh,
        out_type=jax.ShapeDtypeStruct((B, D2), jnp.float32),
        scratch_types=[
            pltpu.VMEM((b_per_w,), jnp.int32),
            pltpu.VMEM((b_per_w, D2), jnp.float32),
            pltpu.SemaphoreType.DMA,
        ],
    )
    def k(tbl_hbm, idx_hbm, out_hbm, idx_v, rows_v, sem):
        wid = lax.axis_index("s") * NC + lax.axis_index("c")
        base = wid * b_per_w

        pltpu.sync_copy(idx_hbm.at[pl.ds(base, b_per_w)], idx_v)
        pltpu.async_copy(tbl_hbm.at[idx_v], rows_v, sem).wait()
        pltpu.sync_copy(rows_v, out_hbm.at[pl.ds(base, b_per_w)])

    return k(table2, idx)


@jax.jit
def kernel(x, policy):
    idx = x.astype(jnp.int32)
    n_states, n_actions = policy.shape
    table2 = _tc_relayout(policy.T)
    return _gather_call(table2, idx, n_actions)[:, :n_actions]


# speedup vs baseline: 1.7454x; 1.5252x over previous
import functools

import jax
import jax.numpy as jnp
from jax import lax
from jax.experimental import pallas as pl
from jax.experimental.pallas import tpu as pltpu
from jax.experimental.pallas import tpu_sc as plsc

_W = 2048


def _tc_relayout(table_t):
    D, V = table_t.shape
    grid = -(-(V // 2) // _W)


    def body(in0_ref, in1_ref, out_ref):
        out_ref[...] = jnp.concatenate(
            [in0_ref[...].T, in1_ref[...].T], axis=1
        )

    return pl.pallas_call(
        body,
        grid=(grid,),
        in_specs=[
            pl.BlockSpec((D, _W), lambda q: (0, 2 * q)),
            pl.BlockSpec(
                (D, _W),
                lambda q: (0, jnp.minimum(2 * q + 1, V // _W)),
            ),
        ],
        out_specs=pl.BlockSpec((_W, 2 * D), lambda q: (q, 0)),
        out_shape=jax.ShapeDtypeStruct((grid * _W, 2 * D), jnp.float32),
    )(table_t, table_t)


def _gather_call(table2, idx):
    V2, D2 = table2.shape
    B = idx.shape[0]
    NC, NS, L = 2, 16, 16
    NW = NC * NS
    b_per_w = B // NW

    mesh = plsc.VectorSubcoreMesh(core_axis_name="c", subcore_axis_name="s")

    @functools.partial(
        pl.kernel,
        mesh=mesh,
        out_type=jax.ShapeDtypeStruct((B, D2), jnp.float32),
        scratch_types=[
            pltpu.VMEM((b_per_w,), jnp.int32),
            pltpu.VMEM((b_per_w,), jnp.int32),
            pltpu.VMEM((b_per_w, D2), jnp.float32),
            pltpu.SemaphoreType.DMA,
        ],
    )
    def k(tbl_hbm, idx_hbm, out_hbm, idx_v, row_v, rows_v, sem):
        wid = lax.axis_index("s") * NC + lax.axis_index("c")
        base = wid * b_per_w

        pltpu.sync_copy(idx_hbm.at[pl.ds(base, b_per_w)], idx_v)

        def pack(c, _):
            xv = idx_v[pl.ds(c * L, L)]
            row_v[pl.ds(c * L, L)] = ((xv >> 12) << 11) | (xv & (_W - 1))
            return ()

        lax.fori_loop(0, b_per_w // L, pack, (), unroll=4)

        pltpu.async_copy(tbl_hbm.at[row_v], rows_v, sem).wait()
        pltpu.sync_copy(rows_v, out_hbm.at[pl.ds(base, b_per_w)])

    return k(table2, idx)


@jax.jit
def kernel(x, policy):
    idx = x.astype(jnp.int32)
    n_states, n_actions = policy.shape
    table2 = _tc_relayout(policy.T)
    pairs = _gather_call(table2, idx)
    hi = (idx >> 11) & 1
    return jnp.where(
        hi[:, None] == 1, pairs[:, n_actions:], pairs[:, :n_actions]
    )


# speedup vs baseline: 2.4098x; 1.3807x over previous
import functools

import jax
import jax.numpy as jnp
from jax import lax
from jax.experimental import pallas as pl
from jax.experimental.pallas import tpu as pltpu
from jax.experimental.pallas import tpu_sc as plsc

_W = 8192


def _tc_relayout(table_t):
    D, V = table_t.shape
    grid = -(-(V // 2) // _W)


    def body(in0_ref, in1_ref, out_ref):
        out_ref[...] = jnp.concatenate(
            [in0_ref[...].T, in1_ref[...].T], axis=1
        )

    return pl.pallas_call(
        body,
        grid=(grid,),
        in_specs=[
            pl.BlockSpec((D, _W), lambda q: (0, 2 * q)),
            pl.BlockSpec(
                (D, _W),
                lambda q: (0, jnp.minimum(2 * q + 1, V // _W)),
            ),
        ],
        out_specs=pl.BlockSpec((_W, 2 * D), lambda q: (q, 0)),
        out_shape=jax.ShapeDtypeStruct((grid * _W, 2 * D), jnp.float32),
    )(table_t, table_t)


def _gather_call(table2, idx):
    V2, D2 = table2.shape
    B = idx.shape[0]
    NC, NS, L = 2, 16, 16
    NW = NC * NS
    b_per_w = B // NW

    mesh = plsc.VectorSubcoreMesh(core_axis_name="c", subcore_axis_name="s")

    @functools.partial(
        pl.kernel,
        mesh=mesh,
        out_type=jax.ShapeDtypeStruct((B, D2), jnp.float32),
        scratch_types=[
            pltpu.VMEM((b_per_w,), jnp.int32),
            pltpu.VMEM((b_per_w,), jnp.int32),
            pltpu.VMEM((b_per_w, D2), jnp.float32),
            pltpu.SemaphoreType.DMA,
        ],
    )
    def k(tbl_hbm, idx_hbm, out_hbm, idx_v, row_v, rows_v, sem):
        wid = lax.axis_index("s") * NC + lax.axis_index("c")
        base = wid * b_per_w

        pltpu.sync_copy(idx_hbm.at[pl.ds(base, b_per_w)], idx_v)

        def pack(c, _):
            xv = idx_v[pl.ds(c * L, L)]
            row_v[pl.ds(c * L, L)] = ((xv >> 14) << 13) | (xv & (_W - 1))
            return ()

        lax.fori_loop(0, b_per_w // L, pack, (), unroll=4)

        pltpu.async_copy(tbl_hbm.at[row_v], rows_v, sem).wait()
        pltpu.sync_copy(rows_v, out_hbm.at[pl.ds(base, b_per_w)])

    return k(table2, idx)


@jax.jit
def kernel(x, policy):
    idx = x.astype(jnp.int32)
    n_states, n_actions = policy.shape
    table2 = _tc_relayout(policy.T)
    pairs = _gather_call(table2, idx)
    hi = (idx >> 13) & 1
    return jnp.where(
        hi[:, None] == 1, pairs[:, n_actions:], pairs[:, :n_actions]
    )


# speedup vs baseline: 2.5457x; 1.0564x over previous
import functools

import jax
import jax.numpy as jnp
from jax import lax
from jax.experimental import pallas as pl
from jax.experimental.pallas import tpu as pltpu
from jax.experimental.pallas import tpu_sc as plsc

_W = 16384


def _tc_relayout(table_t):
    D, V = table_t.shape
    grid = -(-(V // 2) // _W)


    def body(in0_ref, in1_ref, out_ref):
        out_ref[...] = jnp.concatenate(
            [in0_ref[...].T, in1_ref[...].T], axis=1
        )

    return pl.pallas_call(
        body,
        grid=(grid,),
        in_specs=[
            pl.BlockSpec((D, _W), lambda q: (0, 2 * q)),
            pl.BlockSpec(
                (D, _W),
                lambda q: (0, jnp.minimum(2 * q + 1, V // _W)),
            ),
        ],
        out_specs=pl.BlockSpec((_W, 2 * D), lambda q: (q, 0)),
        out_shape=jax.ShapeDtypeStruct((grid * _W, 2 * D), jnp.float32),
    )(table_t, table_t)


def _gather_call(table2, idx):
    V2, D2 = table2.shape
    B = idx.shape[0]
    NC, NS, L = 2, 16, 16
    NW = NC * NS
    b_per_w = B // NW

    mesh = plsc.VectorSubcoreMesh(core_axis_name="c", subcore_axis_name="s")

    @functools.partial(
        pl.kernel,
        mesh=mesh,
        out_type=jax.ShapeDtypeStruct((B, D2), jnp.float32),
        scratch_types=[
            pltpu.VMEM((b_per_w,), jnp.int32),
            pltpu.VMEM((b_per_w,), jnp.int32),
            pltpu.VMEM((b_per_w, D2), jnp.float32),
            pltpu.SemaphoreType.DMA,
        ],
    )
    def k(tbl_hbm, idx_hbm, out_hbm, idx_v, row_v, rows_v, sem):
        wid = lax.axis_index("s") * NC + lax.axis_index("c")
        base = wid * b_per_w

        pltpu.sync_copy(idx_hbm.at[pl.ds(base, b_per_w)], idx_v)

        def pack(c, _):
            xv = idx_v[pl.ds(c * L, L)]
            row_v[pl.ds(c * L, L)] = ((xv >> 15) << 14) | (xv & (_W - 1))
            return ()

        lax.fori_loop(0, b_per_w // L, pack, (), unroll=4)

        pltpu.async_copy(tbl_hbm.at[row_v], rows_v, sem).wait()
        pltpu.sync_copy(rows_v, out_hbm.at[pl.ds(base, b_per_w)])

    return k(table2, idx)


@jax.jit
def kernel(x, policy):
    idx = x.astype(jnp.int32)
    n_states, n_actions = policy.shape
    table2 = _tc_relayout(policy.T)
    pairs = _gather_call(table2, idx)
    hi = (idx >> 14) & 1
    return jnp.where(
        hi[:, None] == 1, pairs[:, n_actions:], pairs[:, :n_actions]
    )
